# baseline (speedup 1.0000x reference)
import jax
import jax.numpy as jnp
from jax.experimental import pallas as pl
from jax.experimental.pallas import tpu as pltpu

_N = 10000
_F = 128
_BM = 400
_NB = _N // _BM
_HM = _BM // 2


def _gcn_kernel(x_ref, w1_ref, b1_ref, w2_ref, b2_ref, ga_ref, gb_ref,
                out_ref, s1_scr, s2_scr):
    i = pl.program_id(0)

    @pl.when(i == 0)
    def _():
        s1_scr[...] = jnp.dot(
            x_ref[...], w1_ref[...], preferred_element_type=jnp.float32
        )

    @pl.when(i < _NB)
    def _():
        acc_a = jnp.dot(
            ga_ref[...], s1_scr[...], preferred_element_type=jnp.float32
        )
        acc_b = jnp.dot(
            gb_ref[...], s1_scr[...], preferred_element_type=jnp.float32
        )
        h_a = jnp.maximum(acc_a + b1_ref[...], 0.0)
        h_b = jnp.maximum(acc_b + b1_ref[...], 0.0)
        base = i * _BM
        s2_scr[pl.ds(base, _HM), :] = jnp.dot(
            h_a, w2_ref[...], preferred_element_type=jnp.float32
        )
        s2_scr[pl.ds(base + _HM, _HM), :] = jnp.dot(
            h_b, w2_ref[...], preferred_element_type=jnp.float32
        )

    @pl.when(i >= _NB)
    def _():
        out_ref[:_HM, :] = jnp.dot(
            ga_ref[...], s2_scr[...], preferred_element_type=jnp.float32
        ) + b2_ref[...]
        out_ref[_HM:, :] = jnp.dot(
            gb_ref[...], s2_scr[...], preferred_element_type=jnp.float32
        ) + b2_ref[...]


def kernel(g, x, W1, b1, W2, b2):
    return pl.pallas_call(
        _gcn_kernel,
        grid=(2 * _NB,),
        in_specs=[
            pl.BlockSpec((_N, _F), lambda i: (0, 0)),
            pl.BlockSpec((_F, _F), lambda i: (0, 0)),
            pl.BlockSpec((1, _F), lambda i: (0, 0)),
            pl.BlockSpec((_F, _F), lambda i: (0, 0)),
            pl.BlockSpec((1, _F), lambda i: (0, 0)),
            pl.BlockSpec((_HM, _N), lambda i: (2 * (i % _NB), 0)),
            pl.BlockSpec((_HM, _N), lambda i: (2 * (i % _NB) + 1, 0)),
        ],
        out_specs=pl.BlockSpec(
            (_BM, _F), lambda i: ((i // _NB) * (i - _NB), 0)
        ),
        out_shape=jax.ShapeDtypeStruct((_N, _F), jnp.float32),
        scratch_shapes=[
            pltpu.VMEM((_N, _F), jnp.float32),
            pltpu.VMEM((_N, _F), jnp.float32),
        ],
        compiler_params=pltpu.CompilerParams(
            dimension_semantics=("arbitrary",),
        ),
    )(x, W1, b1.reshape(1, _F), W2, b2.reshape(1, _F), g, g)
